# baseline (speedup 1.0000x reference)
import functools

import jax
import jax.numpy as jnp
from jax.experimental import pallas as pl

B = 4096
K = 8
N = 30
IN_C = 128
F = 64
C = 5
BB = 128


def _fused_kernel(real_ref, graph_ref, w1_ref, b1_ref, w2_ref, b2_ref,
                  wlin_ref, blin_ref, wheadt_ref, bhead_ref,
                  out1_ref, out2_ref):
    g = graph_ref[...]
    adj = jnp.mean(g, axis=1)

    r = real_ref[...]
    h = jax.lax.dot_general(
        r, w1_ref[...],
        dimension_numbers=(((2,), (0,)), ((), ())),
        preferred_element_type=jnp.float32)

    x = jax.lax.dot_general(
        adj, h,
        dimension_numbers=(((2,), (1,)), ((0,), (0,))),
        preferred_element_type=jnp.float32)
    x = jnp.maximum(x + b1_ref[...].reshape(1, 1, F), 0.0)

    h2 = jax.lax.dot_general(
        x, w2_ref[...],
        dimension_numbers=(((2,), (0,)), ((), ())),
        preferred_element_type=jnp.float32)
    x2 = jax.lax.dot_general(
        adj, h2,
        dimension_numbers=(((2,), (1,)), ((0,), (0,))),
        preferred_element_type=jnp.float32)
    x2 = jnp.maximum(x2 + b2_ref[...].reshape(1, 1, F), 0.0)

    wlin = wlin_ref[...].reshape(1, 1, F)
    blin = blin_ref[0, 0]
    wheadt = wheadt_ref[...]
    bhead = bhead_ref[...]

    def head(xk, out_ref):
        xl = jnp.sum(xk * wlin, axis=2) + blin
        xr = jnp.maximum(xl, 0.0)
        out = jax.lax.dot_general(
            xr, wheadt,
            dimension_numbers=(((1,), (0,)), ((), ())),
            preferred_element_type=jnp.float32)
        out_ref[...] = out + bhead

    head(x, out1_ref)
    head(x2, out2_ref)


@functools.partial(jax.jit, static_argnames=())
def _run(real, graph, W1, b1, W2, b2, Wlin, blin, Whead, bhead):
    grid = (B // BB,)
    out1, out2 = pl.pallas_call(
        _fused_kernel,
        grid=grid,
        in_specs=[
            pl.BlockSpec((BB, N, IN_C), lambda i: (i, 0, 0)),
            pl.BlockSpec((BB, K, N, N), lambda i: (i, 0, 0, 0)),
            pl.BlockSpec((IN_C, F), lambda i: (0, 0)),
            pl.BlockSpec((1, F), lambda i: (0, 0)),
            pl.BlockSpec((F, F), lambda i: (0, 0)),
            pl.BlockSpec((1, F), lambda i: (0, 0)),
            pl.BlockSpec((1, F), lambda i: (0, 0)),
            pl.BlockSpec((1, 1), lambda i: (0, 0)),
            pl.BlockSpec((N, C), lambda i: (0, 0)),
            pl.BlockSpec((1, C), lambda i: (0, 0)),
        ],
        out_specs=[
            pl.BlockSpec((BB, C), lambda i: (i, 0)),
            pl.BlockSpec((BB, C), lambda i: (i, 0)),
        ],
        out_shape=[
            jax.ShapeDtypeStruct((B, C), jnp.float32),
            jax.ShapeDtypeStruct((B, C), jnp.float32),
        ],
    )(real, graph, W1, b1.reshape(1, F), W2, b2.reshape(1, F),
      Wlin.reshape(1, F), blin.reshape(1, 1), Whead.T, bhead.reshape(1, C))
    return out1, out2


def kernel(real, imag, graph, W1, b1, W2, b2, Wlin, blin, Whead, bhead, layer):
    del imag
    out1, out2 = _run(real, graph, W1, b1, W2, b2, Wlin, blin, Whead, bhead)
    return jnp.where(layer > 1, out2, out1)


# speedup vs baseline: 1.4406x; 1.4406x over previous
import functools

import jax
import jax.numpy as jnp
from jax.experimental import pallas as pl

B = 4096
K = 8
N = 30
IN_C = 128
F = 64
C = 5
BB = 128


def _fused_kernel(real_ref, graph_ref, w1_ref, b1_ref, w2_ref, b2_ref,
                  wlin_ref, blin_ref, wheadt_ref, bhead_ref,
                  out1_ref, out2_ref):
    g = graph_ref[...]
    adjf = jnp.mean(g, axis=1)
    adj = adjf.reshape(BB, N, N)

    r = real_ref[...]
    h = jax.lax.dot_general(
        r, w1_ref[...],
        dimension_numbers=(((2,), (0,)), ((), ())),
        preferred_element_type=jnp.float32)

    x = jax.lax.dot_general(
        adj, h,
        dimension_numbers=(((2,), (1,)), ((0,), (0,))),
        preferred_element_type=jnp.float32)
    x = jnp.maximum(x + b1_ref[...].reshape(1, 1, F), 0.0)

    h2 = jax.lax.dot_general(
        x, w2_ref[...],
        dimension_numbers=(((2,), (0,)), ((), ())),
        preferred_element_type=jnp.float32)
    x2 = jax.lax.dot_general(
        adj, h2,
        dimension_numbers=(((2,), (1,)), ((0,), (0,))),
        preferred_element_type=jnp.float32)
    x2 = jnp.maximum(x2 + b2_ref[...].reshape(1, 1, F), 0.0)

    wlin = wlin_ref[...].reshape(1, 1, F)
    blin = blin_ref[0, 0]
    wheadt = wheadt_ref[...]
    bhead = bhead_ref[...]

    def head(xk, out_ref):
        xl = jnp.sum(xk * wlin, axis=2) + blin
        xr = jnp.maximum(xl, 0.0)
        out = jax.lax.dot_general(
            xr, wheadt,
            dimension_numbers=(((1,), (0,)), ((), ())),
            preferred_element_type=jnp.float32)
        out_ref[...] = out + bhead

    head(x, out1_ref)
    head(x2, out2_ref)


@functools.partial(jax.jit, static_argnames=())
def _run(real, graph, W1, b1, W2, b2, Wlin, blin, Whead, bhead):
    grid = (B // BB,)
    out1, out2 = pl.pallas_call(
        _fused_kernel,
        grid=grid,
        in_specs=[
            pl.BlockSpec((BB, N, IN_C), lambda i: (i, 0, 0)),
            pl.BlockSpec((BB, K, N * N), lambda i: (i, 0, 0)),
            pl.BlockSpec((IN_C, F), lambda i: (0, 0)),
            pl.BlockSpec((1, F), lambda i: (0, 0)),
            pl.BlockSpec((F, F), lambda i: (0, 0)),
            pl.BlockSpec((1, F), lambda i: (0, 0)),
            pl.BlockSpec((1, F), lambda i: (0, 0)),
            pl.BlockSpec((1, 1), lambda i: (0, 0)),
            pl.BlockSpec((N, C), lambda i: (0, 0)),
            pl.BlockSpec((1, C), lambda i: (0, 0)),
        ],
        out_specs=[
            pl.BlockSpec((BB, C), lambda i: (i, 0)),
            pl.BlockSpec((BB, C), lambda i: (i, 0)),
        ],
        out_shape=[
            jax.ShapeDtypeStruct((B, C), jnp.float32),
            jax.ShapeDtypeStruct((B, C), jnp.float32),
        ],
    )(real, graph.reshape(B, K, N * N), W1, b1.reshape(1, F), W2, b2.reshape(1, F),
      Wlin.reshape(1, F), blin.reshape(1, 1), Whead.T, bhead.reshape(1, C))
    return out1, out2


def kernel(real, imag, graph, W1, b1, W2, b2, Wlin, blin, Whead, bhead, layer):
    del imag
    out1, out2 = _run(real, graph, W1, b1, W2, b2, Wlin, blin, Whead, bhead)
    return jnp.where(layer > 1, out2, out1)
